# Initial kernel scaffold; baseline (speedup 1.0000x reference)
import functools

import jax
import jax.numpy as jnp
from jax import lax
from jax.experimental import pallas as pl
from jax.experimental.pallas import tpu as pltpu
from jax.experimental.pallas import tpu_sc as plsc

V = 33
D = 38
NC, NS = 2, 16
NW = NC * NS
T = 1024


def _sc_lookup(n_tokens: int):
    per_w = n_tokens // NW
    n_chunks = per_w // T
    mesh = plsc.VectorSubcoreMesh(core_axis_name="c", subcore_axis_name="s")

    @functools.partial(
        pl.kernel,
        out_type=jax.ShapeDtypeStruct((n_tokens * D,), jnp.float32),
        mesh=mesh,
        scratch_types=[
            pltpu.VMEM((D * V,), jnp.float32),
            pltpu.VMEM((T,), jnp.int32),
            pltpu.VMEM((T * D,), jnp.float32),
        ],
    )
    def body(ids_hbm, tabt_hbm, out_hbm, tabt_v, ids_v, out_v):
        wid = lax.axis_index("s") * NC + lax.axis_index("c")
        base = wid * per_w
        pltpu.sync_copy(tabt_hbm, tabt_v)
        stride_v = lax.iota(jnp.int32, (16,)) * D

        def chunk_body(g, carry):
            tok0 = base + g * T
            pltpu.sync_copy(ids_hbm.at[pl.ds(tok0, T)], ids_v)

            def grp_body(gg, c2):
                ids16 = ids_v[pl.ds(gg * 16, 16)]
                sidx0 = stride_v + gg * (16 * D)
                for c in range(D):
                    vals = plsc.load_gather(tabt_v, [ids16 + (c * V)])
                    plsc.store_scatter(out_v, [sidx0 + c], vals)
                return c2

            lax.fori_loop(0, T // 16, grp_body, 0)
            pltpu.sync_copy(out_v, out_hbm.at[pl.ds(tok0 * D, T * D)])
            return carry

        lax.fori_loop(0, n_chunks, chunk_body, 0)

    return body


def kernel(input_ids, table):
    b, l = input_ids.shape
    n_tokens = b * l
    ids_flat = input_ids.reshape(n_tokens).astype(jnp.int32)
    tabt = jnp.transpose(table).reshape(D * V)
    out = _sc_lookup(n_tokens)(ids_flat, tabt)
    return out.reshape(b, l, D)


# speedup vs baseline: 3.1803x; 3.1803x over previous
import functools

import jax
import jax.numpy as jnp
from jax import lax
from jax.experimental import pallas as pl
from jax.experimental.pallas import tpu as pltpu
from jax.experimental.pallas import tpu_sc as plsc

V = 33
D = 38
NC, NS = 2, 16
NW = NC * NS
T = 1024


def _sc_lookup(n_tokens: int):
    per_w = n_tokens // NW
    n_chunks = per_w // T
    mesh = plsc.VectorSubcoreMesh(core_axis_name="c", subcore_axis_name="s")

    @functools.partial(
        pl.kernel,
        out_type=jax.ShapeDtypeStruct((n_tokens * D,), jnp.float32),
        mesh=mesh,
        scratch_types=[
            pltpu.VMEM((D * V,), jnp.float32),
            pltpu.VMEM((T,), jnp.int32),
            pltpu.VMEM((T * D,), jnp.float32),
        ],
        compiler_params=pltpu.CompilerParams(needs_layout_passes=False),
    )
    def body(ids_hbm, tabt_hbm, out_hbm, tabt_v, ids_v, out_v):
        wid = lax.axis_index("s") * NC + lax.axis_index("c")
        base = wid * per_w
        pltpu.sync_copy(tabt_hbm, tabt_v)
        stride_v = lax.iota(jnp.int32, 16) * D

        def chunk_body(g, carry):
            tok0 = base + g * T
            pltpu.sync_copy(ids_hbm.at[pl.ds(tok0, T)], ids_v)

            def grp_body(gg, c2):
                ids16 = ids_v[pl.ds(gg * 16, 16)]
                sidx0 = stride_v + gg * (16 * D)
                for c in range(D):
                    vals = plsc.load_gather(tabt_v, [ids16 + (c * V)])
                    plsc.store_scatter(out_v, [sidx0 + c], vals)
                return c2

            lax.fori_loop(0, T // 16, grp_body, 0)
            pltpu.sync_copy(out_v, out_hbm.at[pl.ds(tok0 * D, T * D)])
            return carry

        lax.fori_loop(0, n_chunks, chunk_body, 0)

    return body


def kernel(input_ids, table):
    b, l = input_ids.shape
    n_tokens = b * l
    ids_flat = input_ids.reshape(n_tokens).astype(jnp.int32)
    tabt = jnp.transpose(table).reshape(D * V)
    out = _sc_lookup(n_tokens)(ids_flat, tabt)
    return out.reshape(b, l, D)


# speedup vs baseline: 3.9501x; 1.2420x over previous
import functools

import jax
import jax.numpy as jnp
from jax import lax
from jax.experimental import pallas as pl
from jax.experimental.pallas import tpu as pltpu
from jax.experimental.pallas import tpu_sc as plsc

V = 33
D = 38
NC, NS = 2, 16
NW = NC * NS
T = 1024


def _sc_lookup(n_tokens: int):
    per_w = n_tokens // NW
    n_chunks = per_w // T
    mesh = plsc.VectorSubcoreMesh(core_axis_name="c", subcore_axis_name="s")

    @functools.partial(
        pl.kernel,
        out_type=jax.ShapeDtypeStruct((n_tokens * D,), jnp.float32),
        mesh=mesh,
        scratch_types=[
            pltpu.VMEM((D * V,), jnp.float32),
            pltpu.VMEM((2 * T,), jnp.int32),
            pltpu.VMEM((2 * T * D,), jnp.float32),
            pltpu.SemaphoreType.DMA,
            pltpu.SemaphoreType.DMA,
            pltpu.SemaphoreType.DMA,
            pltpu.SemaphoreType.DMA,
        ],
        compiler_params=pltpu.CompilerParams(needs_layout_passes=False),
    )
    def body(ids_hbm, tabt_hbm, out_hbm, tabt_v, ids_v, out_v,
             si0, si1, so0, so1):
        wid = lax.axis_index("s") * NC + lax.axis_index("c")
        base = wid * per_w
        pltpu.sync_copy(tabt_hbm, tabt_v)
        stride_v = lax.iota(jnp.int32, 16) * D
        sem_i = (si0, si1)
        sem_o = (so0, so1)

        def ids_copy(g, b):
            return pltpu.make_async_copy(
                ids_hbm.at[pl.ds(base + g * T, T)], ids_v.at[pl.ds(b * T, T)], sem_i[b])

        def out_copy(g, b):
            return pltpu.make_async_copy(
                out_v.at[pl.ds(b * T * D, T * D)], out_hbm.at[pl.ds((base + g * T) * D, T * D)],
                sem_o[b])

        def compute(b):
            ib = ids_v.at[pl.ds(b * T, T)]
            ob = out_v.at[pl.ds(b * T * D, T * D)]

            def grp_body(gg, c2):
                ids16 = ib[pl.ds(gg * 16, 16)]
                sidx0 = stride_v + gg * (16 * D)
                vals = [plsc.load_gather(tabt_v, [ids16 + (c * V)])
                        for c in range(D)]
                for c in range(D):
                    plsc.store_scatter(ob, [sidx0 + c], vals[c])
                return c2

            lax.fori_loop(0, T // 16, grp_body, 0)

        ids_copy(0, 0).start()
        ids_copy(1, 1).start()

        for b in range(2):
            ids_copy(b, b).wait()
            compute(b)
            out_copy(b, b).start()
            ids_copy(b + 2, b).start()

        def pair_body(go, carry):
            for b in range(2):
                g = go * 2 + b
                ids_copy(g, b).wait()
                out_copy(g - 2, b).wait()
                compute(b)
                out_copy(g, b).start()
                gn = lax.rem(g + 2, n_chunks)
                ids_copy(gn, b).start()
            return carry

        lax.fori_loop(1, n_chunks // 2, pair_body, 0)

        out_copy(n_chunks - 2, 0).wait()
        out_copy(n_chunks - 1, 1).wait()
        ids_copy(0, 0).wait()
        ids_copy(1, 1).wait()

    return body


def kernel(input_ids, table):
    b, l = input_ids.shape
    n_tokens = b * l
    ids_flat = input_ids.reshape(n_tokens).astype(jnp.int32)
    tabt = jnp.transpose(table).reshape(D * V)
    out = _sc_lookup(n_tokens)(ids_flat, tabt)
    return out.reshape(b, l, D)
